# Initial kernel scaffold; baseline (speedup 1.0000x reference)
import jax
import jax.numpy as jnp
from jax.experimental import pallas as pl


def kernel(assessment):
    raise NotImplementedError("write your pallas kernel here")


# speedup vs baseline: 37.9410x; 37.9410x over previous
import functools

import jax
import jax.numpy as jnp
from jax import lax
from jax.experimental import pallas as pl
from jax.experimental.pallas import tpu as pltpu
from jax.experimental.pallas import tpu_sc as plsc

PAIRS = 64
NSUB = 16
LANES = 16
GROUP = 256
NEG = -3.0e38
PAD = -1.0e30


def _plan(n):
    chunk_raw = -(-n // NSUB)
    ngrp = -(-chunk_raw // GROUP)
    pg = min(49, ngrp)
    npieces = -(-ngrp // pg)
    chunk = npieces * pg * GROUP
    return npieces, pg, chunk


def _merge16(rv, rp, cv, cp, sort_lo=True):
    cr = lax.rev(cv, (0,))
    cpr = lax.rev(cp, (0,))
    take = rv >= cr
    hi = jnp.where(take, rv, cr)
    hip = jnp.where(take, rp, cpr)
    lo = jnp.where(take, cr, rv)
    lop = jnp.where(take, cpr, rp)
    hi, hip = plsc.sort_key_val(hi, hip, descending=True)
    if sort_lo:
        lo, lop = plsc.sort_key_val(lo, lop, descending=True)
    return hi, hip, lo, lop


def _insert64(rvs, rps, cv, cp):
    out_v, out_p = [], []
    for j in range(4):
        hi, hip, cv, cp = _merge16(rvs[j], rps[j], cv, cp, sort_lo=(j < 3))
        out_v.append(hi)
        out_p.append(hip)
    return tuple(out_v), tuple(out_p)


def _topk_body(npieces, pg, chunk,
               a_hbm, g_hbm, wv_hbm, wi_hbm,
               abuf, gbuf, sbuf, candv, candi, resv, resi):
    row = lax.axis_index("c")
    sid = lax.axis_index("s")
    piece = pg * GROUP
    ngrp = npieces * pg
    lpad = NSUB * chunk
    cap = candv.shape[0] - LANES
    chunk_base = row * lpad + sid * chunk
    iota = lax.iota(jnp.int32, LANES)
    negv = jnp.full((LANES,), NEG, dtype=jnp.float32)

    maccs = (negv,) * LANES
    for p in range(npieces):
        pltpu.sync_copy(a_hbm.at[pl.ds(chunk_base + p * piece, piece)], abuf)
        pltpu.sync_copy(g_hbm.at[pl.ds(chunk_base + p * piece, piece)], gbuf)

        def grp_a(g_, maccs_, p=p):
            off = g_ * GROUP
            new = []
            for j in range(LANES):
                av = abuf[pl.ds(off + j * LANES, LANES)]
                gv = gbuf[pl.ds(off + j * LANES, LANES)]
                s = av + gv
                sbuf[pl.ds(p * piece + off + j * LANES, LANES)] = s
                new.append(jnp.maximum(maccs_[j], s))
            return tuple(new)

        maccs = lax.fori_loop(0, pg, grp_a, maccs)

    rvs = (negv,) * 4
    rps = (iota,) * 4
    for j in range(LANES):
        sv, sp = plsc.sort_key_val(maccs[j], iota, descending=True)
        rvs, rps = _insert64(rvs, rps, sv, sp)
    t = lax.reduce_min(rvs[3], (0,))

    def grp_b(g_, cnt):
        off = g_ * GROUP
        anym = sbuf[pl.ds(off, LANES)] >= t
        for j in range(1, LANES):
            anym = anym | (sbuf[pl.ds(off + j * LANES, LANES)] >= t)

        def slow(c):
            for j in range(LANES):
                s = sbuf[pl.ds(off + j * LANES, LANES)]
                m = s >= t

                def store(c2, s=s, m=m, j=j):
                    npass = lax.reduce_sum(m.astype(jnp.int32), (0,))
                    idxv = sid * chunk + off + j * LANES + iota
                    plsc.store_compressed(candv.at[pl.ds(c2, LANES)], s, mask=m)
                    plsc.store_compressed(candi.at[pl.ds(c2, LANES)], idxv, mask=m)
                    return jnp.minimum(c2 + npass, cap)

                c = lax.cond(jnp.any(m), store, lambda c2: c2, c)
            return c

        return lax.cond(jnp.any(anym), slow, lambda c: c, cnt)

    cnt = lax.fori_loop(0, ngrp, grp_b, jnp.int32(0))

    rvs = (negv,) * 4
    rps = (jnp.zeros((LANES,), jnp.int32),) * 4

    def blk_c(b, carry):
        rvs_, rps_, rmin = carry
        base = b * LANES
        v = candv[pl.ds(base, LANES)]
        ii = candi[pl.ds(base, LANES)]
        v = jnp.where(base + iota < cnt, v, NEG)

        def ins(carry2):
            rvs2, rps2, _ = carry2
            sv, sp = plsc.sort_key_val(v, ii, descending=True)
            rvs3, rps3 = _insert64(rvs2, rps2, sv, sp)
            return rvs3, rps3, lax.reduce_min(rvs3[3], (0,))

        return lax.cond(lax.reduce_max(v, (0,)) > rmin,
                        ins, lambda c: c, (rvs_, rps_, rmin))

    nblk = (cnt + LANES - 1) // LANES
    rvs, rps, _ = lax.fori_loop(0, nblk, blk_c,
                                (rvs, rps, jnp.float32(NEG)))

    for j in range(4):
        resv[pl.ds(j * LANES, LANES)] = rvs[j]
        resi[pl.ds(j * LANES, LANES)] = rps[j]
    pltpu.sync_copy(resv, wv_hbm.at[row, sid])
    pltpu.sync_copy(resi, wi_hbm.at[row, sid])


def _merge_body(lpad,
                wv_hbm, wi_hbm, a_hbm, ov_hbm, oi_hbm,
                mrgv, mrgi, resi, idxb, avalb, sem):
    row = lax.axis_index("c")
    sid = lax.axis_index("s")

    @pl.when(sid == 0)
    def _():
        pltpu.sync_copy(wv_hbm.at[row], mrgv)
        pltpu.sync_copy(wi_hbm.at[row], mrgi)
        rvs = tuple(mrgv[0, pl.ds(j * LANES, LANES)] for j in range(4))
        rps = tuple(mrgi[0, pl.ds(j * LANES, LANES)] for j in range(4))
        rmin = lax.reduce_min(rvs[3], (0,))
        carry = (rvs, rps, rmin)
        for w in range(1, NSUB):
            for j in range(4):
                v = mrgv[w, pl.ds(j * LANES, LANES)]
                ii = mrgi[w, pl.ds(j * LANES, LANES)]

                def ins(c, v=v, ii=ii):
                    rvs2, rps2, _ = c
                    rvs3, rps3 = _insert64(rvs2, rps2, v, ii)
                    return rvs3, rps3, lax.reduce_min(rvs3[3], (0,))

                carry = lax.cond(lax.reduce_max(v, (0,)) > carry[2],
                                 ins, lambda c: c, carry)
        rvs, rps, _ = carry
        for j in range(4):
            idxb[pl.ds(j * LANES, LANES)] = rps[j] + row * lpad
            resi[pl.ds(j * LANES, LANES)] = rps[j]
        pltpu.async_copy(a_hbm.at[idxb], avalb, sem).wait()
        pltpu.sync_copy(avalb, ov_hbm.at[row])
        pltpu.sync_copy(resi, oi_hbm.at[row])


@jax.jit
def kernel(assessment):
    rows, n = assessment.shape
    npieces, pg, chunk = _plan(n)
    lpad = NSUB * chunk
    piece = pg * GROUP
    cap = 4096

    gkey = jax.random.key(42)
    u = jax.random.uniform(gkey, (rows, n), dtype=jnp.float32,
                           minval=1e-20, maxval=1.0)
    gumbel = -jnp.log(-jnp.log(u))

    a_flat = jnp.pad(assessment, ((0, 0), (0, lpad - n)),
                     constant_values=PAD).reshape(-1)
    g_flat = jnp.pad(gumbel, ((0, 0), (0, lpad - n)),
                     constant_values=PAD).reshape(-1)

    mesh = plsc.VectorSubcoreMesh(core_axis_name="c", subcore_axis_name="s",
                                  num_cores=2, num_subcores=NSUB)
    f32, i32 = jnp.float32, jnp.int32
    params = pltpu.CompilerParams(needs_layout_passes=False)

    wv, wi = pl.kernel(
        functools.partial(_topk_body, npieces, pg, chunk),
        out_type=(jax.ShapeDtypeStruct((2, NSUB, PAIRS), f32),
                  jax.ShapeDtypeStruct((2, NSUB, PAIRS), i32)),
        mesh=mesh,
        scratch_types=[
            pltpu.VMEM((piece,), f32),
            pltpu.VMEM((piece,), f32),
            pltpu.VMEM((chunk,), f32),
            pltpu.VMEM((cap + LANES,), f32),
            pltpu.VMEM((cap + LANES,), i32),
            pltpu.VMEM((PAIRS,), f32),
            pltpu.VMEM((PAIRS,), i32),
        ],
        compiler_params=params,
        name="parent_selector_sc_topk",
    )(a_flat, g_flat)

    ov, oi = pl.kernel(
        functools.partial(_merge_body, lpad),
        out_type=(jax.ShapeDtypeStruct((2, PAIRS), f32),
                  jax.ShapeDtypeStruct((2, PAIRS), i32)),
        mesh=mesh,
        scratch_types=[
            pltpu.VMEM((NSUB, PAIRS), f32),
            pltpu.VMEM((NSUB, PAIRS), i32),
            pltpu.VMEM((PAIRS,), i32),
            pltpu.VMEM((PAIRS,), i32),
            pltpu.VMEM((PAIRS,), f32),
            pltpu.SemaphoreType.DMA,
        ],
        compiler_params=params,
        name="parent_selector_sc_merge",
    )(wv, wi, a_flat)
    return ov[0], oi[0], ov[1], oi[1]


# speedup vs baseline: 82.9666x; 2.1867x over previous
import functools

import jax
import jax.numpy as jnp
import numpy as np
from jax import lax
from jax.experimental import pallas as pl
from jax.experimental.pallas import tpu as pltpu
from jax.experimental.pallas import tpu_sc as plsc

PAIRS = 64
NSUB = 16
LANES = 16
GROUP = 256
NEG = -3.0e38
PAD = -1.0e30


_GUMBEL_CACHE = {}


def _gumbel_flat(rows, n, lpad):
    key = (rows, n, lpad)
    if key not in _GUMBEL_CACHE:
        with jax.ensure_compile_time_eval():
            gkey = jax.random.key(42)
            u = jax.random.uniform(gkey, (rows, n), dtype=jnp.float32,
                                   minval=1e-20, maxval=1.0)
            g = -jnp.log(-jnp.log(u))
            gf = jnp.pad(g, ((0, 0), (0, lpad - n)),
                         constant_values=PAD).reshape(-1)
            _GUMBEL_CACHE[key] = np.asarray(gf)
    return _GUMBEL_CACHE[key]


def _plan(n):
    chunk_raw = -(-n // NSUB)
    ngrp = -(-chunk_raw // GROUP)
    pg = min(49, ngrp)
    npieces = -(-ngrp // pg)
    chunk = npieces * pg * GROUP
    return npieces, pg, chunk


def _merge16(rv, rp, cv, cp, sort_lo=True):
    cr = lax.rev(cv, (0,))
    cpr = lax.rev(cp, (0,))
    take = rv >= cr
    hi = jnp.where(take, rv, cr)
    hip = jnp.where(take, rp, cpr)
    lo = jnp.where(take, cr, rv)
    lop = jnp.where(take, cpr, rp)
    hi, hip = plsc.sort_key_val(hi, hip, descending=True)
    if sort_lo:
        lo, lop = plsc.sort_key_val(lo, lop, descending=True)
    return hi, hip, lo, lop


def _insert64(rvs, rps, cv, cp):
    out_v, out_p = [], []
    for j in range(4):
        hi, hip, cv, cp = _merge16(rvs[j], rps[j], cv, cp, sort_lo=(j < 3))
        out_v.append(hi)
        out_p.append(hip)
    return tuple(out_v), tuple(out_p)


def _topk_body(npieces, pg, chunk,
               a_hbm, g_hbm, wv_hbm, wi_hbm,
               abuf, gbuf, sbuf, candv, candi, resv, resi):
    row = lax.axis_index("c")
    sid = lax.axis_index("s")
    piece = pg * GROUP
    ngrp = npieces * pg
    lpad = NSUB * chunk
    cap = candv.shape[0] - LANES
    chunk_base = row * lpad + sid * chunk
    iota = lax.iota(jnp.int32, LANES)
    negv = jnp.full((LANES,), NEG, dtype=jnp.float32)

    maccs = (negv,) * LANES
    for p in range(npieces):
        pltpu.sync_copy(a_hbm.at[pl.ds(chunk_base + p * piece, piece)], abuf)
        pltpu.sync_copy(g_hbm.at[pl.ds(chunk_base + p * piece, piece)], gbuf)

        def grp_a(g_, maccs_, p=p):
            off = g_ * GROUP
            new = []
            for j in range(LANES):
                av = abuf[pl.ds(off + j * LANES, LANES)]
                gv = gbuf[pl.ds(off + j * LANES, LANES)]
                s = av + gv
                sbuf[pl.ds(p * piece + off + j * LANES, LANES)] = s
                new.append(jnp.maximum(maccs_[j], s))
            return tuple(new)

        maccs = lax.fori_loop(0, pg, grp_a, maccs)

    rvs = (negv,) * 4
    rps = (iota,) * 4
    for j in range(LANES):
        sv, sp = plsc.sort_key_val(maccs[j], iota, descending=True)
        rvs, rps = _insert64(rvs, rps, sv, sp)
    t = lax.reduce_min(rvs[3], (0,))

    def grp_b(g_, cnt):
        off = g_ * GROUP
        anym = sbuf[pl.ds(off, LANES)] >= t
        for j in range(1, LANES):
            anym = anym | (sbuf[pl.ds(off + j * LANES, LANES)] >= t)

        def slow(c):
            for j in range(LANES):
                s = sbuf[pl.ds(off + j * LANES, LANES)]
                m = s >= t

                def store(c2, s=s, m=m, j=j):
                    npass = lax.reduce_sum(m.astype(jnp.int32), (0,))
                    idxv = sid * chunk + off + j * LANES + iota
                    plsc.store_compressed(candv.at[pl.ds(c2, LANES)], s, mask=m)
                    plsc.store_compressed(candi.at[pl.ds(c2, LANES)], idxv, mask=m)
                    return jnp.minimum(c2 + npass, cap)

                c = lax.cond(jnp.any(m), store, lambda c2: c2, c)
            return c

        return lax.cond(jnp.any(anym), slow, lambda c: c, cnt)

    cnt = lax.fori_loop(0, ngrp, grp_b, jnp.int32(0))

    rvs = (negv,) * 4
    rps = (jnp.zeros((LANES,), jnp.int32),) * 4

    def blk_c(b, carry):
        rvs_, rps_, rmin = carry
        base = b * LANES
        v = candv[pl.ds(base, LANES)]
        ii = candi[pl.ds(base, LANES)]
        v = jnp.where(base + iota < cnt, v, NEG)

        def ins(carry2):
            rvs2, rps2, _ = carry2
            sv, sp = plsc.sort_key_val(v, ii, descending=True)
            rvs3, rps3 = _insert64(rvs2, rps2, sv, sp)
            return rvs3, rps3, lax.reduce_min(rvs3[3], (0,))

        return lax.cond(lax.reduce_max(v, (0,)) > rmin,
                        ins, lambda c: c, (rvs_, rps_, rmin))

    nblk = (cnt + LANES - 1) // LANES
    rvs, rps, _ = lax.fori_loop(0, nblk, blk_c,
                                (rvs, rps, jnp.float32(NEG)))

    for j in range(4):
        resv[pl.ds(j * LANES, LANES)] = rvs[j]
        resi[pl.ds(j * LANES, LANES)] = rps[j]
    pltpu.sync_copy(resv, wv_hbm.at[row, sid])
    pltpu.sync_copy(resi, wi_hbm.at[row, sid])


def _merge_body(lpad,
                wv_hbm, wi_hbm, a_hbm, ov_hbm, oi_hbm,
                mrgv, mrgi, resi, idxb, avalb, sem):
    row = lax.axis_index("c")
    sid = lax.axis_index("s")

    @pl.when(sid == 0)
    def _():
        pltpu.sync_copy(wv_hbm.at[row], mrgv)
        pltpu.sync_copy(wi_hbm.at[row], mrgi)
        rvs = tuple(mrgv[0, pl.ds(j * LANES, LANES)] for j in range(4))
        rps = tuple(mrgi[0, pl.ds(j * LANES, LANES)] for j in range(4))
        rmin = lax.reduce_min(rvs[3], (0,))
        carry = (rvs, rps, rmin)
        for w in range(1, NSUB):
            for j in range(4):
                v = mrgv[w, pl.ds(j * LANES, LANES)]
                ii = mrgi[w, pl.ds(j * LANES, LANES)]

                def ins(c, v=v, ii=ii):
                    rvs2, rps2, _ = c
                    rvs3, rps3 = _insert64(rvs2, rps2, v, ii)
                    return rvs3, rps3, lax.reduce_min(rvs3[3], (0,))

                carry = lax.cond(lax.reduce_max(v, (0,)) > carry[2],
                                 ins, lambda c: c, carry)
        rvs, rps, _ = carry
        for j in range(4):
            idxb[pl.ds(j * LANES, LANES)] = rps[j] + row * lpad
            resi[pl.ds(j * LANES, LANES)] = rps[j]
        pltpu.async_copy(a_hbm.at[idxb], avalb, sem).wait()
        pltpu.sync_copy(avalb, ov_hbm.at[row])
        pltpu.sync_copy(resi, oi_hbm.at[row])


@jax.jit
def kernel(assessment):
    rows, n = assessment.shape
    npieces, pg, chunk = _plan(n)
    lpad = NSUB * chunk
    piece = pg * GROUP
    cap = 4096

    a_flat = jnp.pad(assessment, ((0, 0), (0, lpad - n)),
                     constant_values=PAD).reshape(-1)
    g_flat = jnp.asarray(_gumbel_flat(rows, n, lpad))

    mesh = plsc.VectorSubcoreMesh(core_axis_name="c", subcore_axis_name="s",
                                  num_cores=2, num_subcores=NSUB)
    f32, i32 = jnp.float32, jnp.int32
    params = pltpu.CompilerParams(needs_layout_passes=False)

    wv, wi = pl.kernel(
        functools.partial(_topk_body, npieces, pg, chunk),
        out_type=(jax.ShapeDtypeStruct((2, NSUB, PAIRS), f32),
                  jax.ShapeDtypeStruct((2, NSUB, PAIRS), i32)),
        mesh=mesh,
        scratch_types=[
            pltpu.VMEM((piece,), f32),
            pltpu.VMEM((piece,), f32),
            pltpu.VMEM((chunk,), f32),
            pltpu.VMEM((cap + LANES,), f32),
            pltpu.VMEM((cap + LANES,), i32),
            pltpu.VMEM((PAIRS,), f32),
            pltpu.VMEM((PAIRS,), i32),
        ],
        compiler_params=params,
        name="parent_selector_sc_topk",
    )(a_flat, g_flat)

    ov, oi = pl.kernel(
        functools.partial(_merge_body, lpad),
        out_type=(jax.ShapeDtypeStruct((2, PAIRS), f32),
                  jax.ShapeDtypeStruct((2, PAIRS), i32)),
        mesh=mesh,
        scratch_types=[
            pltpu.VMEM((NSUB, PAIRS), f32),
            pltpu.VMEM((NSUB, PAIRS), i32),
            pltpu.VMEM((PAIRS,), i32),
            pltpu.VMEM((PAIRS,), i32),
            pltpu.VMEM((PAIRS,), f32),
            pltpu.SemaphoreType.DMA,
        ],
        compiler_params=params,
        name="parent_selector_sc_merge",
    )(wv, wi, a_flat)
    return ov[0], oi[0], ov[1], oi[1]
